# baseline (speedup 1.0000x reference)
import functools

import jax
import jax.numpy as jnp
from jax import lax
from jax.experimental import pallas as pl
from jax.experimental.pallas import tpu as pltpu
from jax.experimental.pallas import tpu_sc as plsc

Q = 1024
K = 100000
D = 64
TOPK = 10
TOPG = 16

QB = 256
KB = 4096
NKB = (K + KB - 1) // KB
KPAD = NKB * KB
GROUP = 16
GPB = KB // GROUP
NG = NKB * GPB
NCAND = TOPG * GROUP
QCB = 32
NEG = float("-inf")
BIGI = 1 << 30


def _ab_body(q_ref, k_ref, aux_ref, outT_ref, ct_ref):
    kb = pl.program_id(1)
    simsT = lax.dot_general(k_ref[...], q_ref[...],
                            (((1,), (1,)), ((), ())),
                            preferred_element_type=jnp.float32)
    aux = aux_ref[...]
    bias = jnp.where(aux == 0.0, -1e38, 0.0)
    x = simsT * aux + bias
    x3 = x.reshape(GPB, GROUP, QB)
    w = GROUP
    while w > 1:
        w //= 2
        x3 = jnp.maximum(x3[:, :w, :], x3[:, w:2 * w, :])
    ct_ref[pl.ds(kb * GPB, GPB), :] = x3.reshape(GPB, QB)

    @pl.when(kb == NKB - 1)
    def _():
        cur = ct_ref[...]
        gidio = lax.broadcasted_iota(jnp.int32, (NG, QB), 0)
        sels = []
        for _t in range(TOPG):
            m = jnp.max(cur, axis=0, keepdims=True)
            sel = jnp.min(jnp.where(cur == m, gidio, BIGI),
                          axis=0, keepdims=True)
            sels.append(sel)
            cur = jnp.where(gidio == sel, NEG, cur)
        outT_ref[...] = jnp.concatenate(sels, axis=0)


_stage_ab = pl.pallas_call(
    _ab_body,
    grid=(Q // QB, NKB),
    in_specs=[
        pl.BlockSpec((QB, D), lambda qb, kb: (qb, 0)),
        pl.BlockSpec((KB, D), lambda qb, kb: (kb, 0)),
        pl.BlockSpec((KB, 1), lambda qb, kb: (kb, 0)),
    ],
    out_specs=pl.BlockSpec((TOPG, QB), lambda qb, kb: (0, qb)),
    out_shape=jax.ShapeDtypeStruct((TOPG, Q), jnp.int32),
    scratch_shapes=[pltpu.VMEM((NG, QB), jnp.float32)],
    compiler_params=pltpu.CompilerParams(
        dimension_semantics=("arbitrary", "arbitrary")),
)


def _c_body(q_ref, gk_ref, kn_ref, ci_ref, qn_ref, vals_ref, idx_ref):
    q = q_ref[...]
    keys3 = gk_ref[...].reshape(QCB * NCAND, D).astype(jnp.bfloat16)
    s2 = lax.dot_general(q, keys3, (((1,), (1,)), ((), ())),
                         preferred_element_type=jnp.float32)
    x = s2.reshape(QCB, QCB, NCAND)
    bi = lax.broadcasted_iota(jnp.int32, (QCB, QCB, NCAND), 0)
    qi = lax.broadcasted_iota(jnp.int32, (QCB, QCB, NCAND), 1)
    x = jnp.where(bi == qi, x, NEG)
    w = QCB
    while w > 1:
        w //= 2
        x = jnp.maximum(x[:, :w, :], x[:, w:2 * w, :])
    dots = x.reshape(QCB, NCAND)
    kn = kn_ref[...]
    qn = qn_ref[...]
    denom = qn * kn + 1e-6
    v = dots * (1.0 / denom)
    ci = ci_ref[...]
    for t in range(TOPK):
        m = jnp.max(v, axis=1, keepdims=True)
        sel = jnp.min(jnp.where(v == m, ci, BIGI), axis=1, keepdims=True)
        v = jnp.where(ci == sel, NEG, v)
        vals_ref[:, t:t + 1] = jnp.where(m != 0.0, m, 0.0)
        idx_ref[:, t:t + 1] = sel


_stage_c = pl.pallas_call(
    _c_body,
    grid=(Q // QCB,),
    in_specs=[
        pl.BlockSpec((QCB, D), lambda i: (i, 0)),
        pl.BlockSpec((QCB, NCAND, D), lambda i: (i, 0, 0)),
        pl.BlockSpec((QCB, NCAND), lambda i: (i, 0)),
        pl.BlockSpec((QCB, NCAND), lambda i: (i, 0)),
        pl.BlockSpec((QCB, 1), lambda i: (i, 0)),
    ],
    out_specs=[
        pl.BlockSpec((QCB, TOPK), lambda i: (i, 0)),
        pl.BlockSpec((QCB, TOPK), lambda i: (i, 0)),
    ],
    out_shape=[
        jax.ShapeDtypeStruct((Q, TOPK), jnp.float32),
        jax.ShapeDtypeStruct((Q, TOPK), jnp.int32),
    ],
)


_NC, _NS = 2, 16
_NW = _NC * _NS
_B = Q * TOPG
_BPW = _B // _NW
_CH = 64
_NCH = _BPW // _CH


def _sc_gather_body(kt_hbm, knt_hbm, idx_hbm, outk_hbm, outn_hbm,
                    idx_v, kb_v, kn_v, sem_k, sem_n):
    wid = lax.axis_index("s") * _NC + lax.axis_index("c")

    def body(c, _):
        base = wid * _BPW + c * _CH
        pltpu.sync_copy(idx_hbm.at[pl.ds(base, _CH)], idx_v)
        ck = pltpu.async_copy(kt_hbm.at[idx_v], kb_v, sem_k)
        cn = pltpu.async_copy(knt_hbm.at[idx_v], kn_v, sem_n)
        ck.wait()
        cn.wait()
        pltpu.sync_copy(kb_v, outk_hbm.at[pl.ds(base, _CH)])
        pltpu.sync_copy(kn_v, outn_hbm.at[pl.ds(base, _CH)])
        return 0

    lax.fori_loop(0, _NCH, body, 0)


@functools.cache
def _sc_gather():
    return functools.partial(
        pl.kernel,
        mesh=plsc.VectorSubcoreMesh(core_axis_name="c", subcore_axis_name="s"),
        out_type=[
            jax.ShapeDtypeStruct((_B, GROUP * D), jnp.float32),
            jax.ShapeDtypeStruct((_B, 128), jnp.float32),
        ],
        scratch_types=[
            pltpu.VMEM((_CH,), jnp.int32),
            pltpu.VMEM((_CH, GROUP * D), jnp.float32),
            pltpu.VMEM((_CH, 128), jnp.float32),
            pltpu.SemaphoreType.DMA,
            pltpu.SemaphoreType.DMA,
        ],
    )(_sc_gather_body)


def kernel(queries, keys, k):
    kn = jnp.sqrt(jnp.sum(keys * keys, axis=1))
    qn = jnp.sqrt(jnp.sum(queries * queries, axis=1))
    aux_col = jnp.pad(1.0 / (kn + 1e-7), (0, KPAD - K)).reshape(KPAD, 1)

    q16 = queries.astype(jnp.bfloat16)
    k16 = keys.astype(jnp.bfloat16)
    k16p = jnp.pad(k16, ((0, KPAD - K), (0, 0)))

    gidT = _stage_ab(q16, k16p, aux_col)
    gid = gidT.T

    kt = keys.reshape(K // GROUP, GROUP * D)
    knt = jnp.pad(kn.reshape(K // GROUP, GROUP), ((0, 0), (0, 128 - GROUP)))
    gkb, gkn = _sc_gather()(kt, knt, gid.reshape(-1))
    gk = gkb.reshape(Q, NCAND, D)
    kn_g = gkn[:, :GROUP].reshape(Q, NCAND)
    cidx = (gid[:, :, None] * GROUP
            + jnp.arange(GROUP, dtype=jnp.int32)).reshape(Q, NCAND)

    vals, idx = _stage_c(q16, gk, kn_g, cidx, qn.reshape(Q, 1))
    vals = jnp.where(jnp.arange(TOPK)[None, :] < k, vals, 0.0)
    return vals, idx


# speedup vs baseline: 1.0503x; 1.0503x over previous
import functools

import jax
import jax.numpy as jnp
from jax import lax
from jax.experimental import pallas as pl
from jax.experimental.pallas import tpu as pltpu
from jax.experimental.pallas import tpu_sc as plsc

Q = 1024
K = 100000
D = 64
TOPK = 10
TOPG = 16

QB = 256
KB = 4096
NKB = (K + KB - 1) // KB
KPAD = NKB * KB
GROUP = 16
GPB = KB // GROUP
NG = NKB * GPB
NCAND = TOPG * GROUP
QCB = 64
NEG = float("-inf")
BIGI = 1 << 30


def _ab_body(q_ref, k_ref, aux_ref, outT_ref, ct_ref):
    kb = pl.program_id(1)
    simsT = lax.dot_general(k_ref[...], q_ref[...],
                            (((1,), (1,)), ((), ())),
                            preferred_element_type=jnp.float32)
    aux = aux_ref[...]
    bias = jnp.where(aux == 0.0, -1e38, 0.0)
    x = simsT * aux + bias
    x3 = x.reshape(GPB, GROUP, QB)
    w = GROUP
    while w > 1:
        w //= 2
        x3 = jnp.maximum(x3[:, :w, :], x3[:, w:2 * w, :])
    ct_ref[pl.ds(kb * GPB, GPB), :] = x3.reshape(GPB, QB)

    @pl.when(kb == NKB - 1)
    def _():
        cur = ct_ref[...]
        gidio = lax.broadcasted_iota(jnp.int32, (NG, QB), 0)
        sels = []
        for _t in range(TOPG):
            m = jnp.max(cur, axis=0, keepdims=True)
            sel = jnp.min(jnp.where(cur == m, gidio, BIGI),
                          axis=0, keepdims=True)
            sels.append(sel)
            cur = jnp.where(gidio == sel, NEG, cur)
        outT_ref[...] = jnp.concatenate(sels, axis=0)


_stage_ab = pl.pallas_call(
    _ab_body,
    grid=(Q // QB, NKB),
    in_specs=[
        pl.BlockSpec((QB, D), lambda qb, kb: (qb, 0)),
        pl.BlockSpec((KB, D), lambda qb, kb: (kb, 0)),
        pl.BlockSpec((KB, 1), lambda qb, kb: (kb, 0)),
    ],
    out_specs=pl.BlockSpec((TOPG, QB), lambda qb, kb: (0, qb)),
    out_shape=jax.ShapeDtypeStruct((TOPG, Q), jnp.int32),
    scratch_shapes=[pltpu.VMEM((NG, QB), jnp.float32)],
    compiler_params=pltpu.CompilerParams(
        dimension_semantics=("arbitrary", "arbitrary")),
)


def _c_body(q_ref, gk_ref, kn_ref, ci_ref, qn_ref, vals_ref, idx_ref):
    q = q_ref[...]
    keys3 = gk_ref[...].reshape(QCB * NCAND, D).astype(jnp.bfloat16)
    s2 = lax.dot_general(q, keys3, (((1,), (1,)), ((), ())),
                         preferred_element_type=jnp.float32)
    x = s2.reshape(QCB, QCB, NCAND)
    bi = lax.broadcasted_iota(jnp.int32, (QCB, QCB, NCAND), 0)
    qi = lax.broadcasted_iota(jnp.int32, (QCB, QCB, NCAND), 1)
    x = jnp.where(bi == qi, x, NEG)
    w = QCB
    while w > 1:
        w //= 2
        x = jnp.maximum(x[:, :w, :], x[:, w:2 * w, :])
    dots = x.reshape(QCB, NCAND)
    kn = kn_ref[...]
    qn = qn_ref[...]
    denom = qn * kn + 1e-6
    v = dots * (1.0 / denom)
    ci = ci_ref[...]
    for t in range(TOPK):
        m = jnp.max(v, axis=1, keepdims=True)
        sel = jnp.min(jnp.where(v == m, ci, BIGI), axis=1, keepdims=True)
        v = jnp.where(ci == sel, NEG, v)
        vals_ref[:, t:t + 1] = jnp.where(m != 0.0, m, 0.0)
        idx_ref[:, t:t + 1] = sel


_stage_c = pl.pallas_call(
    _c_body,
    grid=(Q // QCB,),
    in_specs=[
        pl.BlockSpec((QCB, D), lambda i: (i, 0)),
        pl.BlockSpec((QCB, NCAND, D), lambda i: (i, 0, 0)),
        pl.BlockSpec((QCB, NCAND), lambda i: (i, 0)),
        pl.BlockSpec((QCB, NCAND), lambda i: (i, 0)),
        pl.BlockSpec((QCB, 1), lambda i: (i, 0)),
    ],
    out_specs=[
        pl.BlockSpec((QCB, TOPK), lambda i: (i, 0)),
        pl.BlockSpec((QCB, TOPK), lambda i: (i, 0)),
    ],
    out_shape=[
        jax.ShapeDtypeStruct((Q, TOPK), jnp.float32),
        jax.ShapeDtypeStruct((Q, TOPK), jnp.int32),
    ],
)


_NC, _NS = 2, 16
_NW = _NC * _NS
_B = Q * TOPG
_BPW = _B // _NW
_CH = 64
_NCH = _BPW // _CH


def _sc_gather_body(kt_hbm, knt_hbm, idx_hbm, outk_hbm, outn_hbm,
                    idx_v, kb_v, kn_v, sem_k, sem_n):
    wid = lax.axis_index("s") * _NC + lax.axis_index("c")

    def body(c, _):
        base = wid * _BPW + c * _CH
        pltpu.sync_copy(idx_hbm.at[pl.ds(base, _CH)], idx_v)
        ck = pltpu.async_copy(kt_hbm.at[idx_v], kb_v, sem_k)
        cn = pltpu.async_copy(knt_hbm.at[idx_v], kn_v, sem_n)
        ck.wait()
        cn.wait()
        pltpu.sync_copy(kb_v, outk_hbm.at[pl.ds(base, _CH)])
        pltpu.sync_copy(kn_v, outn_hbm.at[pl.ds(base, _CH)])
        return 0

    lax.fori_loop(0, _NCH, body, 0)


@functools.cache
def _sc_gather():
    return functools.partial(
        pl.kernel,
        mesh=plsc.VectorSubcoreMesh(core_axis_name="c", subcore_axis_name="s"),
        out_type=[
            jax.ShapeDtypeStruct((_B, GROUP * D), jnp.float32),
            jax.ShapeDtypeStruct((_B, 128), jnp.float32),
        ],
        scratch_types=[
            pltpu.VMEM((_CH,), jnp.int32),
            pltpu.VMEM((_CH, GROUP * D), jnp.float32),
            pltpu.VMEM((_CH, 128), jnp.float32),
            pltpu.SemaphoreType.DMA,
            pltpu.SemaphoreType.DMA,
        ],
    )(_sc_gather_body)


def kernel(queries, keys, k):
    kn = jnp.sqrt(jnp.sum(keys * keys, axis=1))
    qn = jnp.sqrt(jnp.sum(queries * queries, axis=1))
    aux_col = jnp.pad(1.0 / (kn + 1e-7), (0, KPAD - K)).reshape(KPAD, 1)

    q16 = queries.astype(jnp.bfloat16)
    k16 = keys.astype(jnp.bfloat16)
    k16p = jnp.pad(k16, ((0, KPAD - K), (0, 0)))

    gidT = _stage_ab(q16, k16p, aux_col)
    gid = gidT.T

    kt = keys.reshape(K // GROUP, GROUP * D)
    knt = jnp.pad(kn.reshape(K // GROUP, GROUP), ((0, 0), (0, 128 - GROUP)))
    gkb, gkn = _sc_gather()(kt, knt, gid.reshape(-1))
    gk = gkb.reshape(Q, NCAND, D)
    kn_g = gkn[:, :GROUP].reshape(Q, NCAND)
    cidx = (gid[:, :, None] * GROUP
            + jnp.arange(GROUP, dtype=jnp.int32)).reshape(Q, NCAND)

    vals, idx = _stage_c(q16, gk, kn_g, cidx, qn.reshape(Q, 1))
    vals = jnp.where(jnp.arange(TOPK)[None, :] < k, vals, 0.0)
    return vals, idx
